# baseline (speedup 1.0000x reference)
import functools

import jax
import jax.numpy as jnp
from jax import lax
from jax.experimental import pallas as pl
from jax.experimental.pallas import tpu as pltpu
from jax.experimental.pallas import tpu_sc as plsc

_CHUNK = 128
_NW = 32
_NB = 5


@functools.cache
def _build(B, D, rows_per_w):
    mesh = plsc.VectorSubcoreMesh(core_axis_name="c", subcore_axis_name="s")
    n_groups = rows_per_w // _NB

    @functools.partial(
        pl.kernel,
        mesh=mesh,
        out_type=jax.ShapeDtypeStruct((B, D), jnp.float32),
        scratch_types=[
            pltpu.VMEM((rows_per_w, _CHUNK), jnp.int32),
            pltpu.VMEM((_NB, _CHUNK, D), jnp.float32),
        ]
        + [pltpu.SemaphoreType.DMA] * (2 * _NB),
    )
    def k(idx_hbm, table_hbm, out_hbm, idx_v, rows_v, *sems):
        gsem = sems[:_NB]
        osem = sems[_NB:]
        wid = lax.axis_index("s") * 2 + lax.axis_index("c")
        row0 = wid * rows_per_w
        pltpu.sync_copy(idx_hbm.at[wid], idx_v)

        def body(g, carry):
            j0 = g * _NB
            gh = [
                pltpu.async_copy(
                    table_hbm.at[idx_v.at[j0 + b]], rows_v.at[b], gsem[b]
                )
                for b in range(_NB)
            ]
            sh = []
            for b in range(_NB):
                gh[b].wait()
                sh.append(
                    pltpu.async_copy(
                        rows_v.at[b],
                        out_hbm.at[pl.ds((row0 + j0 + b) * _CHUNK, _CHUNK)],
                        osem[b],
                    )
                )
            for h in sh:
                h.wait()
            return carry

        lax.fori_loop(0, n_groups, body, 0)

    return k


def kernel(labels, table):
    D = table.shape[1]
    BT, T = labels.shape
    B = BT * T
    idx = labels.astype(jnp.int32).T
    n_rows = B // _CHUNK
    rows_per_w = n_rows // _NW
    idx3 = idx.reshape(_NW, rows_per_w, _CHUNK)
    out = _build(B, D, rows_per_w)(idx3, table)
    return out.reshape(T, BT, D).transpose(1, 0, 2)
